# Initial kernel scaffold; baseline (speedup 1.0000x reference)
import jax
import jax.numpy as jnp
from jax.experimental import pallas as pl


def kernel(x, edge_index, edge_attr, u, batch, eW1, eb1, eW2, eb2, eW3, eb3, nW1, nb1, nW2, nb2, nW3, nb3, gW1, gb1, gW2, gb2, gW3, gb3):
    raise NotImplementedError("write your pallas kernel here")


# speedup vs baseline: 7.5550x; 7.5550x over previous
import functools
import jax
import jax.numpy as jnp
from jax import lax
from jax.experimental import pallas as pl
from jax.experimental.pallas import tpu as pltpu
from jax.experimental.pallas import tpu_sc as plsc

N = 10000
E = 320000
D = 128
DE = 16
G = 64
H = 128
O = 128
DU = 128

NC = 2
NS = 16
NW = NC * NS

CHUNK = 256
NCHUNK = E // CHUNK
KMAX = (NCHUNK + NW - 1) // NW

F32 = jnp.float32


def _prep_body(x_ref, b3_ref, u_ref, eW1s_ref, eW1d_ref, eW1u_ref, eb1_ref,
               nW1x_ref, nW1u_ref, nb1_ref,
               xs2_ref, xd_ref, npre_ref):
    xb = x_ref[...]
    ue = jnp.dot(u_ref[...], eW1u_ref[...], preferred_element_type=F32) + eb1_ref[...]
    un = jnp.dot(u_ref[...], nW1u_ref[...], preferred_element_type=F32) + nb1_ref[...]
    brow = b3_ref[0]
    BN = brow.shape[1]
    onehT = (jnp.broadcast_to(brow, (G, BN)) ==
             lax.broadcasted_iota(jnp.int32, (G, BN), 0)).astype(F32)
    gather_ue = lax.dot_general(onehT, ue, (((0,), (0,)), ((), ())),
                                preferred_element_type=F32)
    gather_un = lax.dot_general(onehT, un, (((0,), (0,)), ((), ())),
                                preferred_element_type=F32)
    xs2_ref[...] = jnp.dot(xb, eW1s_ref[...], preferred_element_type=F32) + gather_ue
    xd_ref[...] = jnp.dot(xb, eW1d_ref[...], preferred_element_type=F32)
    npre_ref[...] = jnp.dot(xb, nW1x_ref[...], preferred_element_type=F32) + gather_un


def _prep(x, batch3, u, eW1s, eW1d, eW1u, eb1, nW1x, nW1u, nb1):
    BN = 1000
    grid = N // BN
    full = lambda shape: pl.BlockSpec(shape, lambda i: (0, 0))
    return pl.pallas_call(
        _prep_body,
        grid=(grid,),
        in_specs=[
            pl.BlockSpec((BN, D), lambda i: (i, 0)),
            pl.BlockSpec((1, 1, BN), lambda i: (i, 0, 0)),
            full((G, DU)),
            full((D, H)), full((D, H)), full((DU, H)), full((1, H)),
            full((D, H)), full((DU, H)), full((1, H)),
        ],
        out_specs=[
            pl.BlockSpec((BN, H), lambda i: (i, 0)),
            pl.BlockSpec((BN, H), lambda i: (i, 0)),
            pl.BlockSpec((BN, H), lambda i: (i, 0)),
        ],
        out_shape=[
            jax.ShapeDtypeStruct((N, H), F32),
            jax.ShapeDtypeStruct((N, H), F32),
            jax.ShapeDtypeStruct((N, H), F32),
        ],
    )(x, batch3, u, eW1s, eW1d, eW1u, eb1, nW1x, nW1u, nb1)


def _sc_gather_body(xs2_hbm, xd_hbm, row2_hbm, col2_hbm, batch_hbm,
                    t_hbm, be2_hbm,
                    row_v, col_v, a_v, b_v, be_v, batch_v, sem):
    cid = lax.axis_index("c")
    sid = lax.axis_index("s")
    wid = sid * NC + cid
    pltpu.sync_copy(batch_hbm, batch_v)

    def chunk_body(k, _):
        c = k * NW + wid

        @pl.when(c < NCHUNK)
        def _():
            r2 = c * 2
            base = c * CHUNK
            pltpu.sync_copy(row2_hbm.at[pl.ds(r2, 2)], row_v)
            pltpu.sync_copy(col2_hbm.at[pl.ds(r2, 2)], col_v)
            d0 = pltpu.async_copy(xs2_hbm.at[row_v.at[0]], a_v.at[pl.ds(0, 128)], sem)
            d1 = pltpu.async_copy(xs2_hbm.at[row_v.at[1]], a_v.at[pl.ds(128, 128)], sem)
            d2 = pltpu.async_copy(xd_hbm.at[col_v.at[0]], b_v.at[pl.ds(0, 128)], sem)
            d3 = pltpu.async_copy(xd_hbm.at[col_v.at[1]], b_v.at[pl.ds(128, 128)], sem)
            d0.wait(); d1.wait(); d2.wait(); d3.wait()

            for j in range(2):
                for l in range(8):
                    sl = pl.ds(l * 16, 16)
                    idx16 = row_v[j, sl]
                    be_v[j, sl] = plsc.load_gather(batch_v, [idx16])

            def add_body(i, _):
                for l in range(8):
                    sl = pl.ds(l * 16, 16)
                    a_v[i, sl] = a_v[i, sl] + b_v[i, sl]
                return 0
            lax.fori_loop(0, CHUNK, add_body, 0)

            pltpu.sync_copy(a_v, t_hbm.at[pl.ds(base, CHUNK)])
            pltpu.sync_copy(be_v, be2_hbm.at[pl.ds(r2, 2)])
        return 0

    lax.fori_loop(0, KMAX, chunk_body, 0)


def _sc_gather(xs2, xd, row2, col2, batch):
    mesh = plsc.VectorSubcoreMesh(core_axis_name="c", subcore_axis_name="s")
    f = pl.kernel(
        _sc_gather_body,
        out_type=[
            jax.ShapeDtypeStruct((E, H), F32),
            jax.ShapeDtypeStruct((E // 128, 128), jnp.int32),
        ],
        mesh=mesh,
        scratch_types=[
            pltpu.VMEM((2, 128), jnp.int32),
            pltpu.VMEM((2, 128), jnp.int32),
            pltpu.VMEM((CHUNK, H), F32),
            pltpu.VMEM((CHUNK, H), F32),
            pltpu.VMEM((2, 128), jnp.int32),
            pltpu.VMEM((N,), jnp.int32),
            pltpu.SemaphoreType.DMA,
        ],
        compiler_params=pltpu.CompilerParams(needs_layout_passes=False),
    )
    return f(xs2, xd, row2, col2, batch)


def _edge_body(t_ref, ea_ref, be_ref, eW1a_ref, eW2_ref, eb2_ref, eW3_ref, eb3_ref,
               eo_ref, ecnt_ref):
    i = pl.program_id(0)
    h1 = jnp.maximum(
        t_ref[...] + jnp.dot(ea_ref[...], eW1a_ref[...], preferred_element_type=F32),
        0.0)
    h2 = jnp.maximum(
        jnp.dot(h1, eW2_ref[...], preferred_element_type=F32) + eb2_ref[...], 0.0)
    eo_ref[...] = jnp.dot(h2, eW3_ref[...], preferred_element_type=F32) + eb3_ref[...]

    beb = be_ref[0]
    RB = beb.shape[0]
    iog = lax.broadcasted_iota(jnp.int32, (G, 128), 0)
    s = jnp.zeros((G, 128), F32)
    for r in range(RB):
        s = s + (jnp.broadcast_to(beb[r:r + 1, :], (G, 128)) == iog).astype(F32)
    contrib = jnp.broadcast_to(jnp.sum(s, axis=1, keepdims=True), (G, 128))

    @pl.when(i == 0)
    def _():
        ecnt_ref[...] = jnp.zeros_like(ecnt_ref)
    ecnt_ref[...] += contrib


def _edge_mlp(t, ea, be2, eW1a, eW2, eb2, eW3, eb3):
    RB = 20
    BE = RB * 128
    grid = E // BE
    be3 = be2.reshape(grid, RB, 128)
    full = lambda shape: pl.BlockSpec(shape, lambda i: (0, 0))
    return pl.pallas_call(
        _edge_body,
        grid=(grid,),
        in_specs=[
            pl.BlockSpec((BE, H), lambda i: (i, 0)),
            pl.BlockSpec((BE, DE), lambda i: (i, 0)),
            pl.BlockSpec((1, RB, 128), lambda i: (i, 0, 0)),
            full((DE, H)), full((H, H)), full((1, H)), full((H, O)), full((1, O)),
        ],
        out_specs=[
            pl.BlockSpec((BE, O), lambda i: (i, 0)),
            pl.BlockSpec((G, 128), lambda i: (0, 0)),
        ],
        out_shape=[
            jax.ShapeDtypeStruct((E, O), F32),
            jax.ShapeDtypeStruct((G, 128), F32),
        ],
    )(t, ea, be3, eW1a, eW2, eb2, eW3, eb3)


def _sc_scatter_body(eo_hbm, col2_hbm, be2_hbm, zeros_hbm,
                     agg0_hbm, agg1_hbm, es0_hbm, es1_hbm,
                     eo_v, col_v, be_v, acc_sh, esum_sh, sem):
    cid = lax.axis_index("c")
    sid = lax.axis_index("s")
    wid = sid * NC + cid

    @pl.when(sid == 0)
    def _():
        pltpu.sync_copy(zeros_hbm, acc_sh)
        pltpu.sync_copy(zeros_hbm.at[pl.ds(0, G)], esum_sh)

    plsc.subcore_barrier()

    def chunk_body(k, _):
        c = k * NW + wid

        @pl.when(c < NCHUNK)
        def _():
            r2 = c * 2
            base = c * CHUNK
            pltpu.sync_copy(col2_hbm.at[pl.ds(r2, 2)], col_v)
            pltpu.sync_copy(be2_hbm.at[pl.ds(r2, 2)], be_v)
            d = pltpu.async_copy(eo_hbm.at[pl.ds(base, CHUNK)], eo_v, sem)
            d.wait()
            for j in range(2):
                rows = eo_v.at[pl.ds(j * 128, 128)]
                pltpu.sync_copy(rows, acc_sh.at[col_v.at[j]], add=True)
                pltpu.sync_copy(rows, esum_sh.at[be_v.at[j]], add=True)
        return 0

    lax.fori_loop(0, KMAX, chunk_body, 0)
    plsc.subcore_barrier()

    ROWS = 1000
    @pl.when(jnp.logical_and(cid == 0, sid < N // ROWS))
    def _():
        pltpu.sync_copy(acc_sh.at[pl.ds(sid * ROWS, ROWS)],
                        agg0_hbm.at[pl.ds(sid * ROWS, ROWS)])

    @pl.when(jnp.logical_and(cid == 1, sid < N // ROWS))
    def _():
        pltpu.sync_copy(acc_sh.at[pl.ds(sid * ROWS, ROWS)],
                        agg1_hbm.at[pl.ds(sid * ROWS, ROWS)])

    @pl.when(jnp.logical_and(cid == 0, sid == 15))
    def _():
        pltpu.sync_copy(esum_sh, es0_hbm)

    @pl.when(jnp.logical_and(cid == 1, sid == 15))
    def _():
        pltpu.sync_copy(esum_sh, es1_hbm)


def _sc_scatter(eo, col2, be2, zeros_n):
    mesh = plsc.VectorSubcoreMesh(core_axis_name="c", subcore_axis_name="s")
    f = pl.kernel(
        _sc_scatter_body,
        out_type=[
            jax.ShapeDtypeStruct((N, O), F32),
            jax.ShapeDtypeStruct((N, O), F32),
            jax.ShapeDtypeStruct((G, O), F32),
            jax.ShapeDtypeStruct((G, O), F32),
        ],
        mesh=mesh,
        scratch_types=[
            pltpu.VMEM((CHUNK, O), F32),
            pltpu.VMEM((2, 128), jnp.int32),
            pltpu.VMEM((2, 128), jnp.int32),
            pltpu.VMEM_SHARED((N, O), F32),
            pltpu.VMEM_SHARED((G, O), F32),
            pltpu.SemaphoreType.DMA,
        ],
    )
    return f(eo, col2, be2, zeros_n)


def _node_body(npre_ref, a0_ref, a1_ref, b3_ref,
               nW1a_ref, nW2_ref, nb2_ref, nW3_ref, nb3_ref,
               u_ref, es0_ref, es1_ref, ecnt_ref,
               gW1u_ref, gW1n_ref, gW1e_ref, gb1_ref,
               gW2_ref, gb2_ref, gW3_ref, gb3_ref,
               nout_ref, uout_ref, ns_acc, ncnt_acc):
    i = pl.program_id(0)
    agg = a0_ref[...] + a1_ref[...]
    nh1 = jnp.maximum(
        npre_ref[...] + jnp.dot(agg, nW1a_ref[...], preferred_element_type=F32), 0.0)
    nh2 = jnp.maximum(
        jnp.dot(nh1, nW2_ref[...], preferred_element_type=F32) + nb2_ref[...], 0.0)
    nout = jnp.dot(nh2, nW3_ref[...], preferred_element_type=F32) + nb3_ref[...]
    nout_ref[...] = nout

    brow = b3_ref[0]
    BN = brow.shape[1]
    onehT = (jnp.broadcast_to(brow, (G, BN)) ==
             lax.broadcasted_iota(jnp.int32, (G, BN), 0)).astype(F32)

    @pl.when(i == 0)
    def _():
        ns_acc[...] = jnp.zeros_like(ns_acc)
        ncnt_acc[...] = jnp.zeros_like(ncnt_acc)

    ns_acc[...] += lax.dot_general(onehT, nout, (((1,), (0,)), ((), ())),
                                   preferred_element_type=F32)
    ncnt_acc[...] += jnp.broadcast_to(
        jnp.sum(onehT, axis=1, keepdims=True), (G, 128))

    @pl.when(i == pl.num_programs(0) - 1)
    def _():
        node_mean = ns_acc[...] / jnp.maximum(ncnt_acc[...], 1.0)
        esum = es0_ref[...] + es1_ref[...]
        edge_mean = esum / jnp.maximum(ecnt_ref[...], 1.0)
        gh1 = jnp.maximum(
            jnp.dot(u_ref[...], gW1u_ref[...], preferred_element_type=F32)
            + jnp.dot(node_mean, gW1n_ref[...], preferred_element_type=F32)
            + jnp.dot(edge_mean, gW1e_ref[...], preferred_element_type=F32)
            + gb1_ref[...], 0.0)
        gh2 = jnp.maximum(
            jnp.dot(gh1, gW2_ref[...], preferred_element_type=F32) + gb2_ref[...], 0.0)
        uout_ref[...] = jnp.dot(gh2, gW3_ref[...], preferred_element_type=F32) + gb3_ref[...]


def _node_global(npre, agg0, agg1, batch3, nW1a, nW2, nb2, nW3, nb3,
                 u, es0, es1, ecnt, gW1u, gW1n, gW1e, gb1, gW2, gb2, gW3, gb3):
    BN = 1000
    grid = N // BN
    full = lambda shape: pl.BlockSpec(shape, lambda i: (0, 0))
    return pl.pallas_call(
        _node_body,
        grid=(grid,),
        in_specs=[
            pl.BlockSpec((BN, H), lambda i: (i, 0)),
            pl.BlockSpec((BN, O), lambda i: (i, 0)),
            pl.BlockSpec((BN, O), lambda i: (i, 0)),
            pl.BlockSpec((1, 1, BN), lambda i: (i, 0, 0)),
            full((O, H)), full((H, H)), full((1, H)), full((H, O)), full((1, O)),
            full((G, DU)), full((G, O)), full((G, O)), full((G, 128)),
            full((DU, H)), full((O, H)), full((O, H)), full((1, H)),
            full((H, H)), full((1, H)), full((H, DU)), full((1, DU)),
        ],
        out_specs=[
            pl.BlockSpec((BN, O), lambda i: (i, 0)),
            pl.BlockSpec((G, DU), lambda i: (0, 0)),
        ],
        out_shape=[
            jax.ShapeDtypeStruct((N, O), F32),
            jax.ShapeDtypeStruct((G, DU), F32),
        ],
        scratch_shapes=[
            pltpu.VMEM((G, 128), F32),
            pltpu.VMEM((G, 128), F32),
        ],
    )(npre, agg0, agg1, batch3, nW1a, nW2, nb2, nW3, nb3,
      u, es0, es1, ecnt, gW1u, gW1n, gW1e, gb1, gW2, gb2, gW3, gb3)


@jax.jit
def kernel(x, edge_index, edge_attr, u, batch,
           eW1, eb1, eW2, eb2, eW3, eb3,
           nW1, nb1, nW2, nb2, nW3, nb3,
           gW1, gb1, gW2, gb2, gW3, gb3):
    row = edge_index[0]
    col = edge_index[1]
    row2 = row.reshape(E // 128, 128)
    col2 = col.reshape(E // 128, 128)
    batch3 = batch.reshape(N // 1000, 1, 1000)

    eW1s, eW1d, eW1a, eW1u = eW1[:D], eW1[D:2 * D], eW1[2 * D:2 * D + DE], eW1[2 * D + DE:]
    nW1x, nW1a, nW1u = nW1[:D], nW1[D:D + O], nW1[D + O:]
    gW1u, gW1n, gW1e = gW1[:DU], gW1[DU:DU + O], gW1[DU + O:]

    r2 = lambda b: b.reshape(1, -1)

    xs2, xd, npre = _prep(x, batch3, u, eW1s, eW1d, eW1u, r2(eb1),
                          nW1x, nW1u, r2(nb1))
    t, be2 = _sc_gather(xs2, xd, row2, col2, batch)
    edge_out, ecnt = _edge_mlp(t, edge_attr, be2, eW1a, eW2, r2(eb2), eW3, r2(eb3))
    zeros_n = jnp.zeros((N, O), F32)
    agg0, agg1, es0, es1 = _sc_scatter(edge_out, col2, be2, zeros_n)
    node_out, u_out = _node_global(
        npre, agg0, agg1, batch3, nW1a, nW2, r2(nb2), nW3, r2(nb3),
        u, es0, es1, ecnt, gW1u, gW1n, gW1e, r2(gb1), gW2, r2(gb2), gW3, r2(gb3))
    return (node_out, edge_out, u_out)


# speedup vs baseline: 7.9495x; 1.0522x over previous
import functools
import jax
import jax.numpy as jnp
from jax import lax
from jax.experimental import pallas as pl
from jax.experimental.pallas import tpu as pltpu
from jax.experimental.pallas import tpu_sc as plsc

N = 10000
E = 320000
D = 128
DE = 16
G = 64
H = 128
O = 128
DU = 128

NC = 2
NS = 16
NW = NC * NS

CHUNK = 256
NCHUNK = E // CHUNK
KMAX = (NCHUNK + NW - 1) // NW

F32 = jnp.float32


def _prep_body(x_ref, b3_ref, u_ref, eW1s_ref, eW1d_ref, eW1u_ref, eb1_ref,
               nW1x_ref, nW1u_ref, nb1_ref,
               xs2_ref, xd_ref, npre_ref):
    xb = x_ref[...]
    ue = jnp.dot(u_ref[...], eW1u_ref[...], preferred_element_type=F32) + eb1_ref[...]
    un = jnp.dot(u_ref[...], nW1u_ref[...], preferred_element_type=F32) + nb1_ref[...]
    brow = b3_ref[0]
    BN = brow.shape[1]
    onehT = (jnp.broadcast_to(brow, (G, BN)) ==
             lax.broadcasted_iota(jnp.int32, (G, BN), 0)).astype(F32)
    gather_ue = lax.dot_general(onehT, ue, (((0,), (0,)), ((), ())),
                                preferred_element_type=F32)
    gather_un = lax.dot_general(onehT, un, (((0,), (0,)), ((), ())),
                                preferred_element_type=F32)
    xs2_ref[...] = jnp.dot(xb, eW1s_ref[...], preferred_element_type=F32) + gather_ue
    xd_ref[...] = jnp.dot(xb, eW1d_ref[...], preferred_element_type=F32)
    npre_ref[...] = jnp.dot(xb, nW1x_ref[...], preferred_element_type=F32) + gather_un


def _prep(x, batch3, u, eW1s, eW1d, eW1u, eb1, nW1x, nW1u, nb1):
    BN = 1000
    grid = N // BN
    full = lambda shape: pl.BlockSpec(shape, lambda i: (0, 0))
    return pl.pallas_call(
        _prep_body,
        grid=(grid,),
        in_specs=[
            pl.BlockSpec((BN, D), lambda i: (i, 0)),
            pl.BlockSpec((1, 1, BN), lambda i: (i, 0, 0)),
            full((G, DU)),
            full((D, H)), full((D, H)), full((DU, H)), full((1, H)),
            full((D, H)), full((DU, H)), full((1, H)),
        ],
        out_specs=[
            pl.BlockSpec((BN, H), lambda i: (i, 0)),
            pl.BlockSpec((BN, H), lambda i: (i, 0)),
            pl.BlockSpec((BN, H), lambda i: (i, 0)),
        ],
        out_shape=[
            jax.ShapeDtypeStruct((N, H), F32),
            jax.ShapeDtypeStruct((N, H), F32),
            jax.ShapeDtypeStruct((N, H), F32),
        ],
    )(x, batch3, u, eW1s, eW1d, eW1u, eb1, nW1x, nW1u, nb1)


def _sc_gather_body(xs2_hbm, xd_hbm, row2_hbm, col2_hbm, batch_hbm,
                    t_hbm, be2_hbm,
                    row_v, col_v, a_v, be_v, batch_v, sem):
    cid = lax.axis_index("c")
    sid = lax.axis_index("s")
    wid = sid * NC + cid
    pltpu.sync_copy(batch_hbm, batch_v)

    def chunk_body(k, _):
        c = k * NW + wid

        @pl.when(c < NCHUNK)
        def _():
            r2 = c * 2
            base = c * CHUNK
            pltpu.sync_copy(row2_hbm.at[pl.ds(r2, 2)], row_v)
            pltpu.sync_copy(col2_hbm.at[pl.ds(r2, 2)], col_v)
            d0 = pltpu.async_copy(xs2_hbm.at[row_v.at[0]], a_v.at[pl.ds(0, 128)], sem)
            d1 = pltpu.async_copy(xs2_hbm.at[row_v.at[1]], a_v.at[pl.ds(128, 128)], sem)
            d0.wait(); d1.wait()
            d2 = pltpu.async_copy(xd_hbm.at[col_v.at[0]], a_v.at[pl.ds(0, 128)], sem,
                                  add=True)
            d3 = pltpu.async_copy(xd_hbm.at[col_v.at[1]], a_v.at[pl.ds(128, 128)], sem,
                                  add=True)

            for j in range(2):
                for l in range(8):
                    sl = pl.ds(l * 16, 16)
                    idx16 = row_v[j, sl]
                    be_v[j, sl] = plsc.load_gather(batch_v, [idx16])

            d2.wait(); d3.wait()

            pltpu.sync_copy(a_v, t_hbm.at[pl.ds(base, CHUNK)])
            pltpu.sync_copy(be_v, be2_hbm.at[pl.ds(r2, 2)])
        return 0

    lax.fori_loop(0, KMAX, chunk_body, 0)


def _sc_gather(xs2, xd, row2, col2, batch):
    mesh = plsc.VectorSubcoreMesh(core_axis_name="c", subcore_axis_name="s")
    f = pl.kernel(
        _sc_gather_body,
        out_type=[
            jax.ShapeDtypeStruct((E, H), F32),
            jax.ShapeDtypeStruct((E // 128, 128), jnp.int32),
        ],
        mesh=mesh,
        scratch_types=[
            pltpu.VMEM((2, 128), jnp.int32),
            pltpu.VMEM((2, 128), jnp.int32),
            pltpu.VMEM((CHUNK, H), F32),
            pltpu.VMEM((2, 128), jnp.int32),
            pltpu.VMEM((N,), jnp.int32),
            pltpu.SemaphoreType.DMA,
        ],
        compiler_params=pltpu.CompilerParams(needs_layout_passes=False),
    )
    return f(xs2, xd, row2, col2, batch)


def _edge_body(t_ref, ea_ref, be_ref, eW1a_ref, eW2_ref, eb2_ref, eW3_ref, eb3_ref,
               eo_ref, ecnt_ref):
    i = pl.program_id(0)
    h1 = jnp.maximum(
        t_ref[...] + jnp.dot(ea_ref[...], eW1a_ref[...], preferred_element_type=F32),
        0.0)
    h2 = jnp.maximum(
        jnp.dot(h1, eW2_ref[...], preferred_element_type=F32) + eb2_ref[...], 0.0)
    eo_ref[...] = jnp.dot(h2, eW3_ref[...], preferred_element_type=F32) + eb3_ref[...]

    beb = be_ref[0]
    RB = beb.shape[0]
    iog = lax.broadcasted_iota(jnp.int32, (G, 128), 0)
    s = jnp.zeros((G, 128), F32)
    for r in range(RB):
        s = s + (jnp.broadcast_to(beb[r:r + 1, :], (G, 128)) == iog).astype(F32)
    contrib = jnp.broadcast_to(jnp.sum(s, axis=1, keepdims=True), (G, 128))

    @pl.when(i == 0)
    def _():
        ecnt_ref[...] = jnp.zeros_like(ecnt_ref)
    ecnt_ref[...] += contrib


def _edge_mlp(t, ea, be2, eW1a, eW2, eb2, eW3, eb3):
    RB = 20
    BE = RB * 128
    grid = E // BE
    be3 = be2.reshape(grid, RB, 128)
    full = lambda shape: pl.BlockSpec(shape, lambda i: (0, 0))
    return pl.pallas_call(
        _edge_body,
        grid=(grid,),
        in_specs=[
            pl.BlockSpec((BE, H), lambda i: (i, 0)),
            pl.BlockSpec((BE, DE), lambda i: (i, 0)),
            pl.BlockSpec((1, RB, 128), lambda i: (i, 0, 0)),
            full((DE, H)), full((H, H)), full((1, H)), full((H, O)), full((1, O)),
        ],
        out_specs=[
            pl.BlockSpec((BE, O), lambda i: (i, 0)),
            pl.BlockSpec((G, 128), lambda i: (0, 0)),
        ],
        out_shape=[
            jax.ShapeDtypeStruct((E, O), F32),
            jax.ShapeDtypeStruct((G, 128), F32),
        ],
    )(t, ea, be3, eW1a, eW2, eb2, eW3, eb3)


def _sc_scatter_body(eo_hbm, col2_hbm, be2_hbm, zeros_hbm,
                     agg0_hbm, agg1_hbm, es0_hbm, es1_hbm,
                     eo_v, col_v, be_v, acc_sh, esum_sh, sem):
    cid = lax.axis_index("c")
    sid = lax.axis_index("s")
    wid = sid * NC + cid

    @pl.when(sid == 0)
    def _():
        pltpu.sync_copy(zeros_hbm, acc_sh)
        pltpu.sync_copy(zeros_hbm.at[pl.ds(0, G)], esum_sh)

    plsc.subcore_barrier()

    def chunk_body(k, _):
        c = k * NW + wid

        @pl.when(c < NCHUNK)
        def _():
            r2 = c * 2
            base = c * CHUNK
            pltpu.sync_copy(col2_hbm.at[pl.ds(r2, 2)], col_v)
            pltpu.sync_copy(be2_hbm.at[pl.ds(r2, 2)], be_v)
            d = pltpu.async_copy(eo_hbm.at[pl.ds(base, CHUNK)], eo_v, sem)
            d.wait()
            for j in range(2):
                rows = eo_v.at[pl.ds(j * 128, 128)]
                pltpu.sync_copy(rows, acc_sh.at[col_v.at[j]], add=True)
                pltpu.sync_copy(rows, esum_sh.at[be_v.at[j]], add=True)
        return 0

    lax.fori_loop(0, KMAX, chunk_body, 0)
    plsc.subcore_barrier()

    ROWS = 1000
    @pl.when(jnp.logical_and(cid == 0, sid < N // ROWS))
    def _():
        pltpu.sync_copy(acc_sh.at[pl.ds(sid * ROWS, ROWS)],
                        agg0_hbm.at[pl.ds(sid * ROWS, ROWS)])

    @pl.when(jnp.logical_and(cid == 1, sid < N // ROWS))
    def _():
        pltpu.sync_copy(acc_sh.at[pl.ds(sid * ROWS, ROWS)],
                        agg1_hbm.at[pl.ds(sid * ROWS, ROWS)])

    @pl.when(jnp.logical_and(cid == 0, sid == 15))
    def _():
        pltpu.sync_copy(esum_sh, es0_hbm)

    @pl.when(jnp.logical_and(cid == 1, sid == 15))
    def _():
        pltpu.sync_copy(esum_sh, es1_hbm)


def _sc_scatter(eo, col2, be2, zeros_n):
    mesh = plsc.VectorSubcoreMesh(core_axis_name="c", subcore_axis_name="s")
    f = pl.kernel(
        _sc_scatter_body,
        out_type=[
            jax.ShapeDtypeStruct((N, O), F32),
            jax.ShapeDtypeStruct((N, O), F32),
            jax.ShapeDtypeStruct((G, O), F32),
            jax.ShapeDtypeStruct((G, O), F32),
        ],
        mesh=mesh,
        scratch_types=[
            pltpu.VMEM((CHUNK, O), F32),
            pltpu.VMEM((2, 128), jnp.int32),
            pltpu.VMEM((2, 128), jnp.int32),
            pltpu.VMEM_SHARED((N, O), F32),
            pltpu.VMEM_SHARED((G, O), F32),
            pltpu.SemaphoreType.DMA,
        ],
    )
    return f(eo, col2, be2, zeros_n)


def _node_body(npre_ref, a0_ref, a1_ref, b3_ref,
               nW1a_ref, nW2_ref, nb2_ref, nW3_ref, nb3_ref,
               u_ref, es0_ref, es1_ref, ecnt_ref,
               gW1u_ref, gW1n_ref, gW1e_ref, gb1_ref,
               gW2_ref, gb2_ref, gW3_ref, gb3_ref,
               nout_ref, uout_ref, ns_acc, ncnt_acc):
    i = pl.program_id(0)
    agg = a0_ref[...] + a1_ref[...]
    nh1 = jnp.maximum(
        npre_ref[...] + jnp.dot(agg, nW1a_ref[...], preferred_element_type=F32), 0.0)
    nh2 = jnp.maximum(
        jnp.dot(nh1, nW2_ref[...], preferred_element_type=F32) + nb2_ref[...], 0.0)
    nout = jnp.dot(nh2, nW3_ref[...], preferred_element_type=F32) + nb3_ref[...]
    nout_ref[...] = nout

    brow = b3_ref[0]
    BN = brow.shape[1]
    onehT = (jnp.broadcast_to(brow, (G, BN)) ==
             lax.broadcasted_iota(jnp.int32, (G, BN), 0)).astype(F32)

    @pl.when(i == 0)
    def _():
        ns_acc[...] = jnp.zeros_like(ns_acc)
        ncnt_acc[...] = jnp.zeros_like(ncnt_acc)

    ns_acc[...] += lax.dot_general(onehT, nout, (((1,), (0,)), ((), ())),
                                   preferred_element_type=F32)
    ncnt_acc[...] += jnp.broadcast_to(
        jnp.sum(onehT, axis=1, keepdims=True), (G, 128))

    @pl.when(i == pl.num_programs(0) - 1)
    def _():
        node_mean = ns_acc[...] / jnp.maximum(ncnt_acc[...], 1.0)
        esum = es0_ref[...] + es1_ref[...]
        edge_mean = esum / jnp.maximum(ecnt_ref[...], 1.0)
        gh1 = jnp.maximum(
            jnp.dot(u_ref[...], gW1u_ref[...], preferred_element_type=F32)
            + jnp.dot(node_mean, gW1n_ref[...], preferred_element_type=F32)
            + jnp.dot(edge_mean, gW1e_ref[...], preferred_element_type=F32)
            + gb1_ref[...], 0.0)
        gh2 = jnp.maximum(
            jnp.dot(gh1, gW2_ref[...], preferred_element_type=F32) + gb2_ref[...], 0.0)
        uout_ref[...] = jnp.dot(gh2, gW3_ref[...], preferred_element_type=F32) + gb3_ref[...]


def _node_global(npre, agg0, agg1, batch3, nW1a, nW2, nb2, nW3, nb3,
                 u, es0, es1, ecnt, gW1u, gW1n, gW1e, gb1, gW2, gb2, gW3, gb3):
    BN = 1000
    grid = N // BN
    full = lambda shape: pl.BlockSpec(shape, lambda i: (0, 0))
    return pl.pallas_call(
        _node_body,
        grid=(grid,),
        in_specs=[
            pl.BlockSpec((BN, H), lambda i: (i, 0)),
            pl.BlockSpec((BN, O), lambda i: (i, 0)),
            pl.BlockSpec((BN, O), lambda i: (i, 0)),
            pl.BlockSpec((1, 1, BN), lambda i: (i, 0, 0)),
            full((O, H)), full((H, H)), full((1, H)), full((H, O)), full((1, O)),
            full((G, DU)), full((G, O)), full((G, O)), full((G, 128)),
            full((DU, H)), full((O, H)), full((O, H)), full((1, H)),
            full((H, H)), full((1, H)), full((H, DU)), full((1, DU)),
        ],
        out_specs=[
            pl.BlockSpec((BN, O), lambda i: (i, 0)),
            pl.BlockSpec((G, DU), lambda i: (0, 0)),
        ],
        out_shape=[
            jax.ShapeDtypeStruct((N, O), F32),
            jax.ShapeDtypeStruct((G, DU), F32),
        ],
        scratch_shapes=[
            pltpu.VMEM((G, 128), F32),
            pltpu.VMEM((G, 128), F32),
        ],
    )(npre, agg0, agg1, batch3, nW1a, nW2, nb2, nW3, nb3,
      u, es0, es1, ecnt, gW1u, gW1n, gW1e, gb1, gW2, gb2, gW3, gb3)


@jax.jit
def kernel(x, edge_index, edge_attr, u, batch,
           eW1, eb1, eW2, eb2, eW3, eb3,
           nW1, nb1, nW2, nb2, nW3, nb3,
           gW1, gb1, gW2, gb2, gW3, gb3):
    row = edge_index[0]
    col = edge_index[1]
    row2 = row.reshape(E // 128, 128)
    col2 = col.reshape(E // 128, 128)
    batch3 = batch.reshape(N // 1000, 1, 1000)

    eW1s, eW1d, eW1a, eW1u = eW1[:D], eW1[D:2 * D], eW1[2 * D:2 * D + DE], eW1[2 * D + DE:]
    nW1x, nW1a, nW1u = nW1[:D], nW1[D:D + O], nW1[D + O:]
    gW1u, gW1n, gW1e = gW1[:DU], gW1[DU:DU + O], gW1[DU + O:]

    r2 = lambda b: b.reshape(1, -1)

    xs2, xd, npre = _prep(x, batch3, u, eW1s, eW1d, eW1u, r2(eb1),
                          nW1x, nW1u, r2(nb1))
    t, be2 = _sc_gather(xs2, xd, row2, col2, batch)
    edge_out, ecnt = _edge_mlp(t, edge_attr, be2, eW1a, eW2, r2(eb2), eW3, r2(eb3))
    zeros_n = jnp.zeros((N, O), F32)
    agg0, agg1, es0, es1 = _sc_scatter(edge_out, col2, be2, zeros_n)
    node_out, u_out = _node_global(
        npre, agg0, agg1, batch3, nW1a, nW2, r2(nb2), nW3, r2(nb3),
        u, es0, es1, ecnt, gW1u, gW1n, gW1e, r2(gb1), gW2, r2(gb2), gW3, r2(gb3))
    return (node_out, edge_out, u_out)
